# baseline (speedup 1.0000x reference)
import functools

import jax
import jax.numpy as jnp
from jax import lax
from jax.experimental import pallas as pl
from jax.experimental.pallas import tpu as pltpu
from jax.experimental.pallas import tpu_sc as plsc

B = 4
S = 128
NQ = B * S
INPUT_DIM = 1024
D3 = 96
DD = 32
MM = 65536
KK = 32
QQ = 8
GS = 128
NG = MM // GS
CHUNK = 4096
NCHUNK = MM // CHUNK
GPC = CHUNK // GS
VCHUNK = 8192
PGCOL = D3

F32_BIG = 3.0e38
I32_BIG = 2**30

SC_CORES = 2
SC_SUBCORES = 16
SC_WORKERS = SC_CORES * SC_SUBCORES
SC_LANES = 16


def _layer_norm(h, g, b):
    mu = jnp.mean(h, axis=-1, keepdims=True)
    var = jnp.mean((h - mu) * (h - mu), axis=-1, keepdims=True)
    return (h - mu) / jnp.sqrt(var + 1e-5) * g + b


def _gelu_exact(h):
    return 0.5 * h * (1.0 + lax.erf(h * (2.0 ** -0.5)))


def _encode_body(x_ref, win_ref, bin_ref, g1_ref, b1_ref, r_ref, q_ref):
    x = x_ref[...]
    w = win_ref[...]
    h = lax.dot_general(x, w, (((1,), (1,)), ((), ())),
                        preferred_element_type=jnp.float32)
    h = h + bin_ref[...]
    h = _layer_norm(h, g1_ref[...], b1_ref[...])
    z = _gelu_exact(h)
    q = lax.dot_general(z, r_ref[...], (((1,), (0,)), ((), ())),
                        preferred_element_type=jnp.float32,
                        precision=lax.Precision.HIGHEST)
    q_ref[...] = q


def _encode(xf, w_in, b_in, g1, b1, rmat):
    return pl.pallas_call(
        _encode_body,
        out_shape=jax.ShapeDtypeStruct((NQ, DD), jnp.float32),
    )(xf, w_in, b_in, g1, b1, rmat)


def _dist_body(q_ref, keys_ref, scale_ref, dist_ref, gmin_ref):
    q = q_ref[...]
    k = keys_ref[...]
    scale = scale_ref[0]
    q2 = jnp.sum(q * q, axis=1, keepdims=True)
    k2 = jnp.sum(k * k, axis=0)
    qb = q.astype(jnp.bfloat16).astype(jnp.float32)
    cross = lax.dot_general(qb, k, (((1,), (0,)), ((), ())),
                            preferred_element_type=jnp.float32)
    d = q2 + k2[None, :] - 2.0 * cross
    d = jnp.maximum(d, 0.0) * scale
    d3 = d.reshape(NQ, GPC, GS)
    dist_ref[...] = d3
    gmin_ref[0] = jnp.min(d3, axis=2)


def _dist(q, keys, scale):
    return pl.pallas_call(
        _dist_body,
        grid=(NCHUNK,),
        in_specs=[
            pl.BlockSpec((NQ, DD), lambda i: (0, 0)),
            pl.BlockSpec((DD, CHUNK), lambda i: (0, i)),
            pl.BlockSpec(memory_space=pltpu.SMEM),
        ],
        out_specs=[
            pl.BlockSpec((NQ, GPC, GS), lambda i: (0, i, 0)),
            pl.BlockSpec((1, NQ, GPC), lambda i: (i, 0, 0)),
        ],
        out_shape=[
            jax.ShapeDtypeStruct((NQ, NG, GS), jnp.float32),
            jax.ShapeDtypeStruct((NCHUNK, NQ, GPC), jnp.float32),
        ],
    )(q, keys, scale)


def _vpad_body(val_ref, qp_ref, out_ref):
    v = val_ref[...].T
    p = jnp.tanh(jnp.tanh(qp_ref[...].T))
    pg = jnp.mean(p, axis=1, keepdims=True)
    pad = jnp.zeros((v.shape[0], GS - D3 - 1), jnp.float32)
    out_ref[...] = jnp.concatenate([v, pg, pad], axis=1)


def _vpad(values_t, quantum_phase_t):
    return pl.pallas_call(
        _vpad_body,
        grid=(MM // VCHUNK,),
        in_specs=[
            pl.BlockSpec((D3, VCHUNK), lambda i: (0, i)),
            pl.BlockSpec((QQ, VCHUNK), lambda i: (0, i)),
        ],
        out_specs=pl.BlockSpec((VCHUNK, GS), lambda i: (i, 0)),
        out_shape=jax.ShapeDtypeStruct((MM, GS), jnp.float32),
    )(values_t, quantum_phase_t)


def _topk_groups_body(a_ref, gsel_ref, flat_ref):
    li = lax.broadcasted_iota(jnp.int32, (NQ, NG), 1)
    ki = lax.broadcasted_iota(jnp.int32, (NQ, KK), 1)

    def step(k, st):
        a, idxs = st
        gi = jnp.argmin(a, axis=1).astype(jnp.int32)
        idxs = jnp.where(ki == k, gi[:, None], idxs)
        a = jnp.where(li == gi[:, None], jnp.float32(F32_BIG), a)
        return a, idxs

    _, idxs = lax.fori_loop(
        0, KK, step, (a_ref[...], jnp.zeros((NQ, KK), jnp.int32)))
    gsel_ref[...] = idxs
    ri = lax.broadcasted_iota(jnp.int32, (NQ, KK), 0)
    flat_ref[...] = idxs + ri * NG


def _topk_groups(gmin):
    return pl.pallas_call(
        _topk_groups_body,
        out_shape=[
            jax.ShapeDtypeStruct((NQ, KK), jnp.int32),
            jax.ShapeDtypeStruct((NQ, KK), jnp.int32),
        ],
    )(gmin)


def _topk_final_body(c_ref, gsel_ref, dtop_ref, msel_ref):
    ia = lax.broadcasted_iota(jnp.int32, (NQ, KK, GS), 1)
    it = lax.broadcasted_iota(jnp.int32, (NQ, KK, GS), 2)
    ki = lax.broadcasted_iota(jnp.int32, (NQ, KK), 1)
    gsel = gsel_ref[...]

    def step(k, st):
        a, dtop, msel = st
        mv2 = jnp.min(a, axis=2)
        j2 = jnp.argmin(a, axis=2).astype(jnp.int32)
        mv = jnp.min(mv2, axis=1, keepdims=True)
        s = jnp.argmin(mv2, axis=1).astype(jnp.int32)[:, None]
        t = jnp.take_along_axis(j2, s, axis=1)
        g = jnp.take_along_axis(gsel, s, axis=1)
        dtop = jnp.where(ki == k, mv, dtop)
        msel = jnp.where(ki == k, g * GS + t, msel)
        hit = (ia == s[:, :, None]) & (it == t[:, :, None])
        a = jnp.where(hit, jnp.float32(F32_BIG), a)
        return a, dtop, msel

    a0 = c_ref[...].reshape(NQ, KK, GS)
    _, dtop, msel = lax.fori_loop(
        0, KK, step,
        (a0, jnp.zeros((NQ, KK), jnp.float32), jnp.zeros((NQ, KK), jnp.int32)))
    dtop_ref[...] = dtop
    msel_ref[...] = msel


def _topk_final(cand, gsel):
    return pl.pallas_call(
        _topk_final_body,
        out_shape=[
            jax.ShapeDtypeStruct((NQ, KK), jnp.float32),
            jax.ShapeDtypeStruct((NQ, KK), jnp.int32),
        ],
    )(cand, gsel)


def _sc_gather_rows(table2d, idx):
    nrows = NQ * KK
    per_w = nrows // SC_WORKERS
    mesh = plsc.VectorSubcoreMesh(core_axis_name="c", subcore_axis_name="s")

    @functools.partial(
        pl.kernel,
        mesh=mesh,
        out_type=jax.ShapeDtypeStruct((nrows, GS), jnp.float32),
        scratch_types=[
            pltpu.VMEM((per_w,), jnp.int32),
            pltpu.VMEM((per_w, GS), jnp.float32),
            pltpu.SemaphoreType.DMA,
        ],
    )
    def k(tab_hbm, idx_hbm, out_hbm, idx_v, rows_v, sem):
        wid = lax.axis_index("s") * SC_CORES + lax.axis_index("c")
        base = wid * per_w
        pltpu.sync_copy(idx_hbm.at[pl.ds(base, per_w)], idx_v)
        pltpu.async_copy(tab_hbm.at[idx_v], rows_v, sem).wait()
        pltpu.sync_copy(rows_v, out_hbm.at[pl.ds(base, per_w)])

    return k(table2d, idx)


def _combine_body(dtop_ref, v_ref, wout_ref, bout_ref, g2_ref, b2_ref,
                  out_ref):
    v = v_ref[...].reshape(NQ, KK, GS)
    pgsel = v[:, :, PGCOL]
    logits = -dtop_ref[...] + 0.05 * pgsel
    mx = jnp.max(logits, axis=1, keepdims=True)
    e = jnp.exp(logits - mx)
    w = e / jnp.sum(e, axis=1, keepdims=True)
    read = jnp.sum(v * w[:, :, None], axis=1)
    read = read[:, :D3]
    read = read.astype(jnp.bfloat16).astype(jnp.float32)
    h = lax.dot_general(read, wout_ref[...], (((1,), (1,)), ((), ())),
                        preferred_element_type=jnp.float32)
    h = h + bout_ref[...]
    h = _layer_norm(h, g2_ref[...], b2_ref[...])
    out_ref[...] = _gelu_exact(h)


def _combine(dtop, vtop, w_out, b_out, g2, b2):
    return pl.pallas_call(
        _combine_body,
        out_shape=jax.ShapeDtypeStruct((NQ, INPUT_DIM), jnp.float32),
    )(dtop, vtop, w_out, b_out, g2, b2)


def kernel(x, keys, values, quantum_phase, ricci_flow, W_in, b_in, ln1_g,
           ln1_b, fractal_weights, W_out, b_out, ln2_g, ln2_b, temperature):
    xf = x.reshape(NQ, INPUT_DIM)

    fw = jax.nn.softmax(fractal_weights, axis=0)
    sfac = jnp.asarray([0.25 ** s for s in range(4)], dtype=jnp.float32)
    scale = (jnp.sum(fw * sfac) / jnp.maximum(temperature, 1e-6)).reshape(1)
    rdiag = jnp.diagonal(ricci_flow)
    rmat = jnp.repeat(jnp.eye(DD, dtype=jnp.float32), 3, axis=0) \
        * (rdiag / 3.0)[None, :]

    q = _encode(xf, W_in, b_in.reshape(1, D3), ln1_g.reshape(1, D3),
                ln1_b.reshape(1, D3), rmat)

    dist3, gmin3 = _dist(q, keys.T, scale)
    gmin = gmin3.transpose(1, 0, 2).reshape(NQ, NG)

    vaug = _vpad(values.T, quantum_phase.T)

    gsel, flat = _topk_groups(gmin)

    cand = _sc_gather_rows(dist3.reshape(NQ * NG, GS), flat.reshape(NQ * KK))

    dtop, msel = _topk_final(cand, gsel)

    vtop = _sc_gather_rows(vaug, msel.reshape(NQ * KK))

    out = _combine(dtop, vtop, W_out, b_out.reshape(1, INPUT_DIM),
                   ln2_g.reshape(1, INPUT_DIM), ln2_b.reshape(1, INPUT_DIM))
    return out.reshape(B, S, INPUT_DIM)


# speedup vs baseline: 1.2578x; 1.2578x over previous
import functools

import jax
import jax.numpy as jnp
from jax import lax
from jax.experimental import pallas as pl
from jax.experimental.pallas import tpu as pltpu
from jax.experimental.pallas import tpu_sc as plsc

B = 4
S = 128
NQ = B * S
INPUT_DIM = 1024
D3 = 96
DD = 32
MM = 65536
KK = 32
QQ = 8
GS = 128
NG = MM // GS
CHUNK = 4096
NCHUNK = MM // CHUNK
GPC = CHUNK // GS
VCHUNK = 8192
PGCOL = D3

F32_BIG = 3.0e38
I32_BIG = 2**30

SC_CORES = 2
SC_SUBCORES = 16
SC_WORKERS = SC_CORES * SC_SUBCORES
SC_LANES = 16


def _layer_norm(h, g, b):
    mu = jnp.mean(h, axis=-1, keepdims=True)
    var = jnp.mean((h - mu) * (h - mu), axis=-1, keepdims=True)
    return (h - mu) / jnp.sqrt(var + 1e-5) * g + b


def _gelu_exact(h):
    return 0.5 * h * (1.0 + lax.erf(h * (2.0 ** -0.5)))


def _encode_body(x_ref, win_ref, bin_ref, g1_ref, b1_ref, r_ref, q_ref):
    x = x_ref[...]
    w = win_ref[...]
    h = lax.dot_general(x, w, (((1,), (1,)), ((), ())),
                        preferred_element_type=jnp.float32)
    h = h + bin_ref[...]
    h = _layer_norm(h, g1_ref[...], b1_ref[...])
    z = _gelu_exact(h)
    q = lax.dot_general(z, r_ref[...], (((1,), (0,)), ((), ())),
                        preferred_element_type=jnp.float32,
                        precision=lax.Precision.HIGHEST)
    q_ref[...] = q


def _encode(xf, w_in, b_in, g1, b1, rmat):
    return pl.pallas_call(
        _encode_body,
        out_shape=jax.ShapeDtypeStruct((NQ, DD), jnp.float32),
    )(xf, w_in, b_in, g1, b1, rmat)


def _dist_body(q_ref, keys_ref, scale_ref, dist_ref, gmin_ref):
    q = q_ref[...]
    k = keys_ref[...]
    scale = scale_ref[0]
    q2 = jnp.sum(q * q, axis=1, keepdims=True)
    k2 = jnp.sum(k * k, axis=0)
    qb = q.astype(jnp.bfloat16).astype(jnp.float32)
    cross = lax.dot_general(qb, k, (((1,), (0,)), ((), ())),
                            preferred_element_type=jnp.float32)
    d = q2 + k2[None, :] - 2.0 * cross
    d = jnp.maximum(d, 0.0) * scale
    d3 = d.reshape(NQ, GPC, GS)
    dist_ref[...] = d3
    gmin_ref[0] = jnp.min(d3, axis=2)


def _dist(q, keys, scale):
    return pl.pallas_call(
        _dist_body,
        grid=(NCHUNK,),
        in_specs=[
            pl.BlockSpec((NQ, DD), lambda i: (0, 0)),
            pl.BlockSpec((DD, CHUNK), lambda i: (0, i)),
            pl.BlockSpec(memory_space=pltpu.SMEM),
        ],
        out_specs=[
            pl.BlockSpec((NQ, GPC, GS), lambda i: (0, i, 0)),
            pl.BlockSpec((1, NQ, GPC), lambda i: (i, 0, 0)),
        ],
        out_shape=[
            jax.ShapeDtypeStruct((NQ, NG, GS), jnp.float32),
            jax.ShapeDtypeStruct((NCHUNK, NQ, GPC), jnp.float32),
        ],
    )(q, keys, scale)


def _vpad_body(val_ref, qp_ref, out_ref):
    v = val_ref[...].T
    p = jnp.tanh(jnp.tanh(qp_ref[...].T))
    pg = jnp.mean(p, axis=1, keepdims=True)
    pad = jnp.zeros((v.shape[0], GS - D3 - 1), jnp.float32)
    out_ref[...] = jnp.concatenate([v, pg, pad], axis=1)


def _vpad(values_t, quantum_phase_t):
    return pl.pallas_call(
        _vpad_body,
        grid=(MM // VCHUNK,),
        in_specs=[
            pl.BlockSpec((D3, VCHUNK), lambda i: (0, i)),
            pl.BlockSpec((QQ, VCHUNK), lambda i: (0, i)),
        ],
        out_specs=pl.BlockSpec((VCHUNK, GS), lambda i: (i, 0)),
        out_shape=jax.ShapeDtypeStruct((MM, GS), jnp.float32),
    )(values_t, quantum_phase_t)


def _topk_groups_body(a_ref, gsel_ref, flat_ref):
    li = lax.broadcasted_iota(jnp.int32, (NQ, NG), 1)
    ki = lax.broadcasted_iota(jnp.int32, (NQ, KK), 1)

    def step(k, st):
        a, idxs = st
        mv = jnp.min(a, axis=1, keepdims=True)
        sel = jnp.where(a == mv, li, jnp.int32(I32_BIG))
        gi = jnp.min(sel, axis=1)
        idxs = jnp.where(ki == k, gi[:, None], idxs)
        a = jnp.where(li == gi[:, None], jnp.float32(F32_BIG), a)
        return a, idxs

    _, idxs = lax.fori_loop(
        0, KK, step, (a_ref[...], jnp.zeros((NQ, KK), jnp.int32)))
    gsel_ref[...] = idxs
    ri = lax.broadcasted_iota(jnp.int32, (NQ, KK), 0)
    flat_ref[...] = idxs + ri * NG


def _topk_groups(gmin):
    return pl.pallas_call(
        _topk_groups_body,
        out_shape=[
            jax.ShapeDtypeStruct((NQ, KK), jnp.int32),
            jax.ShapeDtypeStruct((NQ, KK), jnp.int32),
        ],
    )(gmin)


def _topk_final_body(c_ref, gsel_ref, dtop_ref, msel_ref):
    ia = lax.broadcasted_iota(jnp.int32, (NQ, KK, GS), 1)
    it = lax.broadcasted_iota(jnp.int32, (NQ, KK, GS), 2)
    lj = ia * GS + it
    ki = lax.broadcasted_iota(jnp.int32, (NQ, KK), 1)

    def step(k, st):
        a, dtop, jsel = st
        mv = jnp.min(a, axis=(1, 2), keepdims=True)
        sel = jnp.where(a == mv, lj, jnp.int32(I32_BIG))
        ji = jnp.min(sel, axis=(1, 2), keepdims=True)
        dtop = jnp.where(ki == k, mv.reshape(NQ, 1), dtop)
        jsel = jnp.where(ki == k, ji.reshape(NQ, 1), jsel)
        a = jnp.where(lj == ji, jnp.float32(F32_BIG), a)
        return a, dtop, jsel

    a0 = c_ref[...].reshape(NQ, KK, GS)
    _, dtop, jsel = lax.fori_loop(
        0, KK, step,
        (a0, jnp.zeros((NQ, KK), jnp.float32), jnp.zeros((NQ, KK), jnp.int32)))
    dtop_ref[...] = dtop
    slot = lax.shift_right_logical(jsel, 7)
    t = jnp.bitwise_and(jsel, jnp.int32(GS - 1))
    g = jnp.take_along_axis(gsel_ref[...], slot, axis=1)
    msel_ref[...] = g * GS + t


def _topk_final(cand, gsel):
    return pl.pallas_call(
        _topk_final_body,
        out_shape=[
            jax.ShapeDtypeStruct((NQ, KK), jnp.float32),
            jax.ShapeDtypeStruct((NQ, KK), jnp.int32),
        ],
    )(cand, gsel)


def _sc_gather_rows(table2d, idx):
    nrows = NQ * KK
    per_w = nrows // SC_WORKERS
    mesh = plsc.VectorSubcoreMesh(core_axis_name="c", subcore_axis_name="s")

    @functools.partial(
        pl.kernel,
        mesh=mesh,
        out_type=jax.ShapeDtypeStruct((nrows, GS), jnp.float32),
        scratch_types=[
            pltpu.VMEM((per_w,), jnp.int32),
            pltpu.VMEM((per_w, GS), jnp.float32),
            pltpu.SemaphoreType.DMA,
        ],
    )
    def k(tab_hbm, idx_hbm, out_hbm, idx_v, rows_v, sem):
        wid = lax.axis_index("s") * SC_CORES + lax.axis_index("c")
        base = wid * per_w
        pltpu.sync_copy(idx_hbm.at[pl.ds(base, per_w)], idx_v)
        pltpu.async_copy(tab_hbm.at[idx_v], rows_v, sem).wait()
        pltpu.sync_copy(rows_v, out_hbm.at[pl.ds(base, per_w)])

    return k(table2d, idx)


def _combine_body(dtop_ref, v_ref, wout_ref, bout_ref, g2_ref, b2_ref,
                  out_ref):
    v = v_ref[...].reshape(NQ, KK, GS)
    pgsel = v[:, :, PGCOL]
    logits = -dtop_ref[...] + 0.05 * pgsel
    mx = jnp.max(logits, axis=1, keepdims=True)
    e = jnp.exp(logits - mx)
    w = e / jnp.sum(e, axis=1, keepdims=True)
    read = jnp.sum(v * w[:, :, None], axis=1)
    read = read[:, :D3]
    read = read.astype(jnp.bfloat16).astype(jnp.float32)
    h = lax.dot_general(read, wout_ref[...], (((1,), (1,)), ((), ())),
                        preferred_element_type=jnp.float32)
    h = h + bout_ref[...]
    h = _layer_norm(h, g2_ref[...], b2_ref[...])
    out_ref[...] = _gelu_exact(h)


def _combine(dtop, vtop, w_out, b_out, g2, b2):
    return pl.pallas_call(
        _combine_body,
        out_shape=jax.ShapeDtypeStruct((NQ, INPUT_DIM), jnp.float32),
    )(dtop, vtop, w_out, b_out, g2, b2)


def kernel(x, keys, values, quantum_phase, ricci_flow, W_in, b_in, ln1_g,
           ln1_b, fractal_weights, W_out, b_out, ln2_g, ln2_b, temperature):
    xf = x.reshape(NQ, INPUT_DIM)

    fw = jax.nn.softmax(fractal_weights, axis=0)
    sfac = jnp.asarray([0.25 ** s for s in range(4)], dtype=jnp.float32)
    scale = (jnp.sum(fw * sfac) / jnp.maximum(temperature, 1e-6)).reshape(1)
    rdiag = jnp.diagonal(ricci_flow)
    rmat = jnp.repeat(jnp.eye(DD, dtype=jnp.float32), 3, axis=0) \
        * (rdiag / 3.0)[None, :]

    q = _encode(xf, W_in, b_in.reshape(1, D3), ln1_g.reshape(1, D3),
                ln1_b.reshape(1, D3), rmat)

    dist3, gmin3 = _dist(q, keys.T, scale)
    gmin = gmin3.transpose(1, 0, 2).reshape(NQ, NG)

    vaug = _vpad(values.T, quantum_phase.T)

    gsel, flat = _topk_groups(gmin)

    cand = _sc_gather_rows(dist3.reshape(NQ * NG, GS), flat.reshape(NQ * KK))

    dtop, msel = _topk_final(cand, gsel)

    vtop = _sc_gather_rows(vaug, msel.reshape(NQ * KK))

    out = _combine(dtop, vtop, W_out, b_out.reshape(1, INPUT_DIM),
                   ln2_g.reshape(1, INPUT_DIM), ln2_b.reshape(1, INPUT_DIM))
    return out.reshape(B, S, INPUT_DIM)
